# speedup vs baseline: 1.5757x; 1.5757x over previous
import functools

import numpy as np
import jax
import jax.numpy as jnp
from jax import lax
from jax.experimental import pallas as pl
from jax.experimental.pallas import tpu as pltpu
from jax.experimental.pallas import tpu_sc as plsc

_B = 4
_L = 2048
_D = 768
_NC = 2
_NS = 16
_NW = _NC * _NS
_PW = _L // _NW
_CD = _D // 16


def _pe_table() -> np.ndarray:
    pos = np.arange(_L, dtype=np.float32).reshape(-1, 1)
    exponent = np.arange(0, _D, 2, dtype=np.float32).reshape(1, -1) / np.float32(_D)
    X = (pos / np.power(np.float32(10000.0), exponent)).astype(np.float32)
    pe = np.zeros((_L, _D), dtype=np.float32)
    pe[:, 0::2] = np.sin(X)
    pe[:, 1::2] = np.cos(X)
    return pe


_MESH = plsc.VectorSubcoreMesh(core_axis_name="c", subcore_axis_name="s")


@functools.partial(
    pl.kernel,
    mesh=_MESH,
    out_type=jax.ShapeDtypeStruct((_B * _L, _D), jnp.float32),
    scratch_types=[
        pltpu.VMEM((_B, _PW), jnp.int32),
        pltpu.VMEM((_PW, _D), jnp.float32),
        pltpu.VMEM((_PW, _D), jnp.float32),
        pltpu.SemaphoreType.DMA,
    ],
)
def _emb_kernel(x_hbm, table_hbm, pe_hbm, out_hbm, idx_v, pe_v, rows_v, sem):
    wid = lax.axis_index("s") * _NC + lax.axis_index("c")
    pos0 = wid * _PW

    pltpu.sync_copy(pe_hbm.at[pl.ds(pos0, _PW)], pe_v)
    for b in range(_B):
        pltpu.sync_copy(x_hbm.at[pl.ds(b * _L + pos0, _PW)], idx_v.at[b])

    for b in range(_B):
        pltpu.async_copy(table_hbm.at[idx_v.at[b]], rows_v, sem).wait()

        def _add_row(r, carry):
            for c in range(_CD):
                sl = pl.ds(c * 16, 16)
                rows_v[r, sl] = rows_v[r, sl] + pe_v[r, sl]
            return carry

        lax.fori_loop(0, _PW, _add_row, 0)
        pltpu.sync_copy(rows_v, out_hbm.at[pl.ds(b * _L + pos0, _PW)])


def kernel(x, table):
    pe = jnp.asarray(_pe_table())
    x_flat = x.reshape(-1).astype(jnp.int32)
    out = _emb_kernel(x_flat, table, pe)
    return out.reshape(_B, _L, _D)
